# Initial kernel scaffold; baseline (speedup 1.0000x reference)
import jax
import jax.numpy as jnp
from jax.experimental import pallas as pl


def kernel(xyz, points, w0, b0, g0, beta0, w1, b1, g1, beta1, w2, b2, g2, beta2):
    raise NotImplementedError("write your pallas kernel here")


# speedup vs baseline: 1.4733x; 1.4733x over previous
import functools

import jax
import jax.numpy as jnp
from jax.experimental import pallas as pl

B, N = 8, 4096
S, K = 512, 64
RADIUS = 0.4
C_IN = 131
MLP_CH = [128, 128, 256]
EPS = 1e-5
S_BLK = 8
R_BLK = 1024


def _fps_kernel(x_ref, out_ref):
    x = x_ref[0]
    iota = jax.lax.broadcasted_iota(jnp.int32, (N, 1), 0)

    def body(i, carry):
        dmin, far = carry
        c = x_ref[0, pl.ds(far, 1), :]
        out_ref[0, pl.ds(i, 1), :] = c
        d = jnp.sum((x - c) ** 2, axis=1, keepdims=True)
        dmin = jnp.minimum(dmin, d)
        m = jnp.max(dmin)
        cand = jnp.where(dmin == m, iota, N)
        far2 = jnp.min(cand).astype(jnp.int32)
        return dmin, far2

    init = (jnp.full((N, 1), 1e10, jnp.float32), jnp.int32(0))
    jax.lax.fori_loop(0, S, body, init)


def _group_kernel(xyzc_ref, feat_ref, nx_ref, out_ref):
    xc = xyzc_ref[0]
    f = feat_ref[0]
    nx = nx_ref[0]
    cross = jax.lax.dot_general(
        nx, xc, (((1,), (0,)), ((), ())), preferred_element_type=jnp.float32)
    nsq = jnp.sum(nx * nx, axis=1, keepdims=True)
    xsq = jnp.sum(xc * xc, axis=0, keepdims=True)
    dist = (-2.0 * cross + nsq) + xsq
    mask = dist <= RADIUS * RADIUS
    iota_n = jax.lax.broadcasted_iota(jnp.int32, (S_BLK, N), 1)
    cols = []
    m = mask
    for k in range(K):
        sel = jnp.where(m, iota_n, N)
        c = jnp.min(sel, axis=1, keepdims=True)
        cols.append(c)
        m = m & (iota_n != c)
    idx = jnp.concatenate(cols, axis=1)
    first = idx[:, 0:1]
    idx = jnp.where(idx == N, first, idx)
    iota3 = jax.lax.broadcasted_iota(jnp.int32, (S_BLK, K, N), 2)
    oh = (idx[:, :, None] == iota3).astype(jnp.float32).reshape(S_BLK * K, N)
    g = jax.lax.dot_general(
        oh, f, (((1,), (0,)), ((), ())), preferred_element_type=jnp.float32)
    nxp = jnp.concatenate(
        [nx, jnp.zeros((S_BLK, C_IN - 3), jnp.float32)], axis=1)
    subs = jnp.broadcast_to(nxp[:, None, :], (S_BLK, K, C_IN))
    out_ref[0] = g - subs.reshape(S_BLK * K, C_IN)


def _layer_kernel(x_ref, w_ref, b_ref, sc_ref, sh_ref, y_ref, s1_ref, s2_ref,
                  *, act):
    x = x_ref[0]
    if act:
        x = jnp.maximum(x * sc_ref[...] + sh_ref[...], 0.0)
    y = jax.lax.dot_general(
        x, w_ref[...], (((1,), (1,)), ((), ())),
        preferred_element_type=jnp.float32) + b_ref[...]
    y_ref[0] = y
    cout = y.shape[-1]
    s1_ref[0, 0] = jnp.broadcast_to(jnp.sum(y, axis=0, keepdims=True), (8, cout))
    s2_ref[0, 0] = jnp.broadcast_to(jnp.sum(y * y, axis=0, keepdims=True), (8, cout))


def _final_kernel(y_ref, sc_ref, sh_ref, out_ref):
    y = y_ref[0]
    h = jnp.maximum(y * sc_ref[...] + sh_ref[...], 0.0)
    h = h.reshape(R_BLK // K, K, h.shape[-1])
    out_ref[0] = jnp.max(h, axis=1)


def _run_layer(x, w, b, scale, shift, act):
    Bq, R, Cin = x.shape
    Cout = w.shape[0]
    nblk = R // R_BLK
    y, s1, s2 = pl.pallas_call(
        functools.partial(_layer_kernel, act=act),
        grid=(Bq, nblk),
        in_specs=[
            pl.BlockSpec((1, R_BLK, Cin), lambda b_, r: (b_, r, 0)),
            pl.BlockSpec((Cout, Cin), lambda b_, r: (0, 0)),
            pl.BlockSpec((1, Cout), lambda b_, r: (0, 0)),
            pl.BlockSpec((1, Cin), lambda b_, r: (0, 0)),
            pl.BlockSpec((1, Cin), lambda b_, r: (0, 0)),
        ],
        out_specs=[
            pl.BlockSpec((1, R_BLK, Cout), lambda b_, r: (b_, r, 0)),
            pl.BlockSpec((1, 1, 8, Cout), lambda b_, r: (b_, r, 0, 0)),
            pl.BlockSpec((1, 1, 8, Cout), lambda b_, r: (b_, r, 0, 0)),
        ],
        out_shape=[
            jax.ShapeDtypeStruct((Bq, R, Cout), jnp.float32),
            jax.ShapeDtypeStruct((Bq, nblk, 8, Cout), jnp.float32),
            jax.ShapeDtypeStruct((Bq, nblk, 8, Cout), jnp.float32),
        ],
    )(x, w, b.reshape(1, -1), scale.reshape(1, -1), shift.reshape(1, -1))
    return y, s1, s2


def _bn_affine(s1, s2, g, beta, count):
    mean = jnp.sum(s1, axis=(0, 1))[0] / count
    var = jnp.sum(s2, axis=(0, 1))[0] / count - mean * mean
    scale = g / jnp.sqrt(var + EPS)
    shift = beta - mean * scale
    return scale, shift


def kernel(xyz, points, w0, b0, g0, beta0, w1, b1, g1, beta1, w2, b2, g2, beta2):
    xyz_t = jnp.transpose(xyz, (0, 2, 1))
    pts_t = jnp.transpose(points, (0, 2, 1))
    feat = jnp.concatenate([xyz_t, pts_t], axis=2)

    new_xyz = pl.pallas_call(
        _fps_kernel,
        grid=(B,),
        in_specs=[pl.BlockSpec((1, N, 3), lambda b_: (b_, 0, 0))],
        out_specs=pl.BlockSpec((1, S, 3), lambda b_: (b_, 0, 0)),
        out_shape=jax.ShapeDtypeStruct((B, S, 3), jnp.float32),
    )(xyz_t)

    x = pl.pallas_call(
        _group_kernel,
        grid=(B, S // S_BLK),
        in_specs=[
            pl.BlockSpec((1, 3, N), lambda b_, s_: (b_, 0, 0)),
            pl.BlockSpec((1, N, C_IN), lambda b_, s_: (b_, 0, 0)),
            pl.BlockSpec((1, S_BLK, 3), lambda b_, s_: (b_, s_, 0)),
        ],
        out_specs=pl.BlockSpec((1, S_BLK * K, C_IN), lambda b_, s_: (b_, s_, 0)),
        out_shape=jax.ShapeDtypeStruct((B, S * K, C_IN), jnp.float32),
    )(xyz, feat, new_xyz)

    count = float(B * S * K)
    one = jnp.ones((C_IN,), jnp.float32)
    zero = jnp.zeros((C_IN,), jnp.float32)
    y0, s1, s2 = _run_layer(x, w0, b0, one, zero, act=False)
    sc0, sh0 = _bn_affine(s1, s2, g0, beta0, count)
    y1, s1, s2 = _run_layer(y0, w1, b1, sc0, sh0, act=True)
    sc1, sh1 = _bn_affine(s1, s2, g1, beta1, count)
    y2, s1, s2 = _run_layer(y1, w2, b2, sc1, sh1, act=True)
    sc2, sh2 = _bn_affine(s1, s2, g2, beta2, count)

    feats = pl.pallas_call(
        _final_kernel,
        grid=(B, (S * K) // R_BLK),
        in_specs=[
            pl.BlockSpec((1, R_BLK, MLP_CH[2]), lambda b_, r: (b_, r, 0)),
            pl.BlockSpec((1, MLP_CH[2]), lambda b_, r: (0, 0)),
            pl.BlockSpec((1, MLP_CH[2]), lambda b_, r: (0, 0)),
        ],
        out_specs=pl.BlockSpec((1, R_BLK // K, MLP_CH[2]), lambda b_, r: (b_, r, 0)),
        out_shape=jax.ShapeDtypeStruct((B, S, MLP_CH[2]), jnp.float32),
    )(y2, sc2.reshape(1, -1), sh2.reshape(1, -1))

    return jnp.transpose(new_xyz, (0, 2, 1)), jnp.transpose(feats, (0, 2, 1))
